# Initial kernel scaffold; baseline (speedup 1.0000x reference)
import jax
import jax.numpy as jnp
from jax.experimental import pallas as pl


def kernel(logits, actions):
    raise NotImplementedError("write your pallas kernel here")


# speedup vs baseline: 1.7455x; 1.7455x over previous
import jax
import jax.numpy as jnp
from jax.experimental import pallas as pl
from jax.experimental.pallas import tpu as pltpu

ROWS = 64
COLS = 100000
CHUNK = 8192
NCHUNK = (COLS + CHUNK - 1) // CHUNK
SUB = CHUNK // 128
NEG = float("-inf")
BIG = 2**31 - 1


def _body(a_ref, x_ref, lp_ref, mode_ref, vm, vs, vi, ga):
    i = pl.program_id(0)

    @pl.when(i == 0)
    def _init():
        vm[...] = jnp.full((ROWS, 128), NEG, jnp.float32)
        vs[...] = jnp.zeros((ROWS, 128), jnp.float32)
        vi[...] = jnp.full((ROWS, 128), BIG, jnp.int32)
        ga[...] = jnp.zeros((ROWS, 128), jnp.float32)

    x3 = x_ref[...].reshape(ROWS, SUB, 128)
    col3 = (i * CHUNK
            + jax.lax.broadcasted_iota(jnp.int32, (ROWS, SUB, 128), 1) * 128
            + jax.lax.broadcasted_iota(jnp.int32, (ROWS, SUB, 128), 2))
    valid = col3 < COLS
    xm = jnp.where(valid, x3, NEG)

    cm = jnp.max(xm, axis=1)
    hit = xm == cm[:, None, :]
    ci = jnp.min(jnp.where(hit, col3, BIG), axis=1)

    m_old = vm[...]
    upd = cm > m_old
    nm = jnp.where(upd, cm, m_old)
    vi[...] = jnp.where(upd, ci, vi[...])
    vs[...] = vs[...] * jnp.exp(m_old - nm) + jnp.sum(
        jnp.exp(xm - nm[:, None, :]), axis=1)
    vm[...] = nm

    a = a_ref[...]
    ga[...] += jnp.sum(jnp.where(col3 == a[:, :, None], xm, 0.0), axis=1)

    @pl.when(i == NCHUNK - 1)
    def _fin():
        vmf = vm[...]
        m = jnp.max(vmf, axis=1, keepdims=True)
        s = jnp.sum(vs[...] * jnp.exp(vmf - m), axis=1, keepdims=True)
        idx = jnp.min(jnp.where(vmf == m, vi[...], BIG), axis=1, keepdims=True)
        gv = jnp.sum(ga[...], axis=1, keepdims=True)
        lp_ref[...] = gv - m - jnp.log(s)
        mode_ref[...] = idx


def kernel(logits, actions):
    actions = actions.astype(jnp.int32)
    lp, mode = pl.pallas_call(
        _body,
        grid=(NCHUNK,),
        in_specs=[
            pl.BlockSpec((ROWS, 1), lambda i: (0, 0)),
            pl.BlockSpec((ROWS, CHUNK), lambda i: (0, i)),
        ],
        out_specs=[
            pl.BlockSpec((ROWS, 1), lambda i: (0, 0)),
            pl.BlockSpec((ROWS, 1), lambda i: (0, 0)),
        ],
        out_shape=[
            jax.ShapeDtypeStruct((ROWS, 1), jnp.float32),
            jax.ShapeDtypeStruct((ROWS, 1), jnp.int32),
        ],
        scratch_shapes=[
            pltpu.VMEM((ROWS, 128), jnp.float32),
            pltpu.VMEM((ROWS, 128), jnp.float32),
            pltpu.VMEM((ROWS, 128), jnp.int32),
            pltpu.VMEM((ROWS, 128), jnp.float32),
        ],
        compiler_params=pltpu.CompilerParams(
            dimension_semantics=("arbitrary",)),
    )(actions, logits)
    return lp, mode
